# baseline (speedup 1.0000x reference)
import functools
import jax
import jax.numpy as jnp
from jax import lax
from jax.experimental import pallas as pl
from jax.experimental.pallas import tpu as pltpu
from jax.experimental.pallas import tpu_sc as plsc

_G = 64
_NG = 26
_W = _G * _NG
_B = 16384
_NC, _NS = 2, 16
_NW = _NC * _NS
_RPW = _B // _NW
_CH = 32
_NCHUNK = _RPW // _CH

_mesh = plsc.VectorSubcoreMesh(core_axis_name="c", subcore_axis_name="s")


@functools.partial(
    pl.kernel,
    out_type=jax.ShapeDtypeStruct((_B, _W), jnp.float32),
    mesh=_mesh,
    scratch_types=[
        pltpu.VMEM((2, _CH, _W), jnp.float32),
        pltpu.SemaphoreType.DMA,
        pltpu.SemaphoreType.DMA,
        pltpu.SemaphoreType.DMA,
        pltpu.SemaphoreType.DMA,
    ],
    compiler_params=pltpu.CompilerParams(use_tc_tiling_on_sc=False),
)
def _sc_permute(in_hbm, out_hbm, buf, sem_in0, sem_in1, sem_out0, sem_out1):
    wid = lax.axis_index("s") * _NC + lax.axis_index("c")
    row0 = wid * _RPW
    sem_in = (sem_in0, sem_in1)
    sem_out = (sem_out0, sem_out1)

    def in_copies(c, b):
        r = row0 + c * _CH
        return [
            pltpu.make_async_copy(
                in_hbm.at[pl.ds(r, _CH), pl.ds(_G * (_NG - 1 - j), _G)],
                buf.at[b, :, pl.ds(_G * j, _G)],
                sem_in[b],
            )
            for j in range(_NG)
        ]

    def out_copy(c, b):
        r = row0 + c * _CH
        return pltpu.make_async_copy(buf.at[b], out_hbm.at[pl.ds(r, _CH)], sem_out[b])

    for b in range(2):
        for cp in in_copies(b, b):
            cp.start()

    @pl.loop(0, _NCHUNK, step=2)
    def _pair(k):
        for b in range(2):
            c = k + b
            for cp in in_copies(c, b):
                cp.wait()
            out_copy(c, b).start()
            out_copy(c, b).wait()

            @pl.when(c + 2 < _NCHUNK)
            def _():
                for cp in in_copies(c + 2, b):
                    cp.start()


def kernel(pooled_embs):
    return _sc_permute(pooled_embs)


# speedup vs baseline: 3.2965x; 3.2965x over previous
import functools
import jax
import jax.numpy as jnp
from jax import lax
from jax.experimental import pallas as pl
from jax.experimental.pallas import tpu as pltpu
from jax.experimental.pallas import tpu_sc as plsc

_G = 64
_NG = 26
_W = _G * _NG
_B = 16384
_NC, _NS = 2, 16
_NW = _NC * _NS
_RPW = _B // _NW
_CH = 32
_NCHUNK = _RPW // _CH
_L = 16

_mesh = plsc.VectorSubcoreMesh(core_axis_name="c", subcore_axis_name="s")


@functools.partial(
    pl.kernel,
    out_type=jax.ShapeDtypeStruct((_B, _W), jnp.float32),
    mesh=_mesh,
    scratch_types=[
        pltpu.VMEM((2, _CH, _W), jnp.float32),
        pltpu.SemaphoreType.DMA,
        pltpu.SemaphoreType.DMA,
        pltpu.SemaphoreType.DMA,
        pltpu.SemaphoreType.DMA,
    ],
    compiler_params=pltpu.CompilerParams(use_tc_tiling_on_sc=False),
)
def _sc_permute(in_hbm, out_hbm, buf, sem_in0, sem_in1, sem_out0, sem_out1):
    wid = lax.axis_index("s") * _NC + lax.axis_index("c")
    row0 = wid * _RPW
    sem_in = (sem_in0, sem_in1)
    sem_out = (sem_out0, sem_out1)

    def in_copy(c, b):
        r = row0 + c * _CH
        return pltpu.make_async_copy(in_hbm.at[pl.ds(r, _CH)], buf.at[b], sem_in[b])

    def out_copy(c, b):
        r = row0 + c * _CH
        return pltpu.make_async_copy(buf.at[b], out_hbm.at[pl.ds(r, _CH)], sem_out[b])

    def permute(b):
        @pl.loop(0, _CH)
        def _row(r):
            for g in range(_NG // 2):
                o1 = _G * g
                o2 = _G * (_NG - 1 - g)
                for i in range(_G // _L):
                    s1 = pl.ds(o1 + _L * i, _L)
                    s2 = pl.ds(o2 + _L * i, _L)
                    a = buf[b, r, s1]
                    z = buf[b, r, s2]
                    buf[b, r, s2] = a
                    buf[b, r, s1] = z

    in_copy(0, 0).start()
    in_copy(1, 1).start()

    @pl.loop(0, _NCHUNK, step=2)
    def _pair(k):
        for b in range(2):
            c = k + b
            in_copy(c, b).wait()
            permute(b)
            out_copy(c, b).start()

        @pl.when(k + 2 < _NCHUNK)
        def _():
            for b in range(2):
                out_copy(k + b, b).wait()
                in_copy(k + 2 + b, b).start()

    out_copy(_NCHUNK - 2, 0).wait()
    out_copy(_NCHUNK - 1, 1).wait()


def kernel(pooled_embs):
    return _sc_permute(pooled_embs)
